# baseline (speedup 1.0000x reference)
import functools

import jax
import jax.numpy as jnp
from jax import lax
from jax.experimental import pallas as pl
from jax.experimental.pallas import tpu as pltpu
from jax.experimental.pallas import tpu_sc as plsc

B = 16384
D = 32
V = 1000000


def _sc_fused(uid, iid, ut_t, it_t):
    info = plsc.get_sparse_core_info()
    nc, ns = info.num_cores, info.num_subcores
    nw = nc * ns
    bpw = B // nw
    ng = bpw // 16
    mesh = plsc.VectorSubcoreMesh(core_axis_name="c", subcore_axis_name="s")

    @functools.partial(
        pl.kernel,
        mesh=mesh,
        compiler_params=pltpu.CompilerParams(use_tc_tiling_on_sc=False),
        out_type=jax.ShapeDtypeStruct((B,), jnp.float32),
        scratch_types=[
            pltpu.VMEM((bpw,), jnp.int32),
            pltpu.VMEM((bpw,), jnp.int32),
            pltpu.VMEM((D, bpw), jnp.float32),
            pltpu.VMEM((D, bpw), jnp.float32),
            pltpu.VMEM((bpw,), jnp.float32),
            pltpu.SemaphoreType.DMA,
            pltpu.SemaphoreType.DMA,
        ],
    )
    def k(uid_hbm, iid_hbm, ut_hbm, it_hbm, out_hbm,
          uid_v, iid_v, du_v, di_v, out_v, semu, semi):
        wid = lax.axis_index("s") * nc + lax.axis_index("c")
        base = wid * bpw
        pltpu.sync_copy(uid_hbm.at[pl.ds(base, bpw)], uid_v)
        pltpu.sync_copy(iid_hbm.at[pl.ds(base, bpw)], iid_v)

        for d in range(D):
            pltpu.async_copy(ut_hbm.at[d].at[uid_v], du_v.at[d], semu)
            pltpu.async_copy(it_hbm.at[d].at[iid_v], di_v.at[d], semi)

        pltpu.make_async_copy(ut_hbm.at[0, pl.ds(0, D * bpw)], du_v, semu).wait()
        pltpu.make_async_copy(it_hbm.at[0, pl.ds(0, D * bpw)], di_v, semi).wait()

        def dot(g, _):
            acc = jnp.zeros((16,), jnp.float32)
            for d in range(D):
                acc += du_v[d, pl.ds(g * 16, 16)] * di_v[d, pl.ds(g * 16, 16)]
            out_v[pl.ds(g * 16, 16)] = acc
            return 0

        lax.fori_loop(0, ng, dot, 0)
        pltpu.sync_copy(out_v, out_hbm.at[pl.ds(base, bpw)])

    return k(uid, iid, ut_t, it_t)


def kernel(input_userID, input_itemID, user_table, item_table):
    uid = input_userID.astype(jnp.int32)
    iid = input_itemID.astype(jnp.int32)
    return _sc_fused(uid, iid, user_table.T, item_table.T)


# speedup vs baseline: 5.6318x; 5.6318x over previous
import functools

import jax
import jax.numpy as jnp
from jax import lax
from jax.experimental import pallas as pl
from jax.experimental.pallas import tpu as pltpu
from jax.experimental.pallas import tpu_sc as plsc

B = 16384
D = 32
V = 1000000


def _sc_fused(uid, iid, ut, it):
    info = plsc.get_sparse_core_info()
    nc, ns = info.num_cores, info.num_subcores
    nw = nc * ns
    bpw = B // nw
    ng = bpw // 16
    mesh = plsc.VectorSubcoreMesh(core_axis_name="c", subcore_axis_name="s")

    @functools.partial(
        pl.kernel,
        mesh=mesh,
        compiler_params=pltpu.CompilerParams(
            use_tc_tiling_on_sc=False, needs_layout_passes=False),
        out_type=jax.ShapeDtypeStruct((B,), jnp.float32),
        scratch_types=[
            pltpu.VMEM((bpw,), jnp.int32),
            pltpu.VMEM((bpw,), jnp.int32),
            pltpu.VMEM((bpw, D), jnp.float32),
            pltpu.VMEM((bpw, D), jnp.float32),
            pltpu.VMEM((bpw,), jnp.float32),
            pltpu.SemaphoreType.DMA,
            pltpu.SemaphoreType.DMA,
        ],
    )
    def k(uid_hbm, iid_hbm, ut_hbm, it_hbm, out_hbm,
          uid_v, iid_v, du_v, di_v, out_v, semu, semi):
        wid = lax.axis_index("s") * nc + lax.axis_index("c")
        base = wid * bpw
        pltpu.sync_copy(uid_hbm.at[pl.ds(base, bpw)], uid_v)
        pltpu.sync_copy(iid_hbm.at[pl.ds(base, bpw)], iid_v)

        cu = pltpu.async_copy(ut_hbm.at[uid_v], du_v, semu)
        ci = pltpu.async_copy(it_hbm.at[iid_v], di_v, semi)
        cu.wait()
        ci.wait()

        def dot(g, _):
            rows = g * 16 + lax.iota(jnp.int32, 16)
            acc = jnp.zeros((16,), jnp.float32)
            for d in range(D):
                dvec = jnp.full((16,), d, jnp.int32)
                u = plsc.load_gather(du_v, [rows, dvec])
                i = plsc.load_gather(di_v, [rows, dvec])
                acc += u * i
            out_v[pl.ds(g * 16, 16)] = acc
            return 0

        lax.fori_loop(0, ng, dot, 0)
        pltpu.sync_copy(out_v, out_hbm.at[pl.ds(base, bpw)])

    return k(uid, iid, ut, it)


def kernel(input_userID, input_itemID, user_table, item_table):
    uid = input_userID.astype(jnp.int32)
    iid = input_itemID.astype(jnp.int32)
    return _sc_fused(uid, iid, user_table, item_table)
